# Initial kernel scaffold; baseline (speedup 1.0000x reference)
import functools

import jax
import jax.numpy as jnp
from jax import lax
from jax.experimental import pallas as pl
from jax.experimental.pallas import tpu as pltpu
from jax.experimental.pallas import tpu_sc as plsc

R = 131072
S = 64
NS = 49
NOUT = 113
HIST_PAD = 0.01
EPS = 1e-05

_info = plsc.get_sparse_core_info()
NCORES = _info.num_cores
NSUB = _info.num_subcores
NWORK = NCORES * NSUB
RAYS_PER_W = R // NWORK
K = 64
NCHUNK = RAYS_PER_W // K


def _body(w_hbm, eb_hbm, nr_hbm, fr_hbm, u_hbm,
          o_se, o_ee, o_sm, o_em,
          w_v, eb_v, nr_v, fr_v, u_v, cdf_v, a_v, h2_v,
          st_e, en_e, st_m, en_m):
    wid = lax.axis_index("s") * NCORES + lax.axis_index("c")
    lanes = lax.iota(jnp.int32, 16)
    ones_i = jnp.ones((16,), jnp.int32)
    zeros_i = jnp.zeros((16,), jnp.int32)

    pltpu.sync_copy(u_hbm, u_v)
    u_vec = [u_v[pl.ds(16 * c, 16)] for c in range(4)]
    jvec = [lanes + 16 * c for c in range(4)]
    ivec = jvec
    jmask = [None, None, None, jvec[3] < NS]

    def chunk_body(cix, carry):
        base = wid * RAYS_PER_W + cix * K
        pltpu.sync_copy(w_hbm.at[pl.ds(base, K), :], w_v)
        pltpu.sync_copy(eb_hbm.at[pl.ds(base, K), :], eb_v)
        pltpu.sync_copy(nr_hbm.at[pl.ds(base, K)], nr_v)
        pltpu.sync_copy(fr_hbm.at[pl.ds(base, K)], fr_v)

        def ray_body(k, rcarry):
            row = lax.broadcast(k, (16,))
            w = [w_v[k, pl.ds(16 * c, 16)] + HIST_PAD for c in range(4)]
            s_pre = [jnp.sum(w[c]) for c in range(4)]
            total = (s_pre[0] + s_pre[1]) + (s_pre[2] + s_pre[3])
            padding = jnp.maximum(EPS - total, 0.0)
            wadj = padding * (1.0 / S)
            wsum = total + padding
            inv = 1.0 / wsum
            carry_s = 0.0
            cdf = []
            for c in range(4):
                wc = w[c] + wadj
                cum = plsc.cumsum(wc) + carry_s
                carry_s = carry_s + s_pre[c] + 16.0 * wadj
                cdf.append(jnp.minimum(1.0, cum * inv))
                cdf_v[pl.ds(16 * c, 16)] = cdf[c]
            a_v[pl.ds(0, 16)] = jnp.where(lanes == 0, 1, 0)
            for c in range(1, 4):
                a_v[pl.ds(16 * c, 16)] = zeros_i
            for c in range(4):
                x = jnp.float32(NS) * cdf[c] - 0.5
                ti = x.astype(jnp.int32)
                j0 = ti + (ti.astype(jnp.float32) < x).astype(jnp.int32)
                j0 = jnp.clip(j0, 0, NS)
                plsc.addupdate_scatter(a_v, [j0], ones_i)
            carry_i = jnp.int32(0)
            below = []
            for c in range(4):
                av = a_v[pl.ds(16 * c, 16)]
                bc = plsc.cumsum(av) + carry_i - 1
                carry_i = carry_i + jnp.sum(av)
                below.append(jnp.minimum(bc, S - 1))
            near = nr_v[k]
            far = fr_v[k]
            scale = far - near
            h2_v[pl.ds(0, 16)] = zeros_i
            for c in range(1, 4):
                h2_v[pl.ds(16 * c, 16)] = zeros_i
            sva = []
            sve = []
            rank_s = []
            for c in range(4):
                b = below[c]
                g0 = plsc.load_gather(cdf_v, [jnp.maximum(b - 1, 0)])
                cdf_g0 = jnp.where(b == 0, 0.0, g0)
                cdf_g1 = plsc.load_gather(cdf_v, [b])
                bins_g0 = plsc.load_gather(eb_v, [row, b])
                bins_g1 = plsc.load_gather(eb_v, [row, b + 1])
                t = (u_vec[c] - cdf_g0) / (cdf_g1 - cdf_g0)
                t = jnp.clip(t, 0.0, 1.0)
                sv = bins_g0 + t * (bins_g1 - bins_g0)
                sva.append(sv)
                sve.append(near + sv * scale)
                rank_s.append(jvec[c] + b + 1)
                plsc.addupdate_scatter(h2_v, [b], ones_i, mask=jmask[c])
            carry_i = jnp.int32(0)
            for c in range(4):
                hv = h2_v[pl.ds(16 * c, 16)]
                excl = plsc.cumsum(hv) - hv + carry_i
                carry_i = carry_i + jnp.sum(hv)
                rank_eb = ivec[c] + excl
                ebv = eb_v[k, pl.ds(16 * c, 16)]
                ebe = near + ebv * scale
                plsc.store_scatter(st_m, [row, rank_eb], ebv)
                plsc.store_scatter(st_e, [row, rank_eb], ebe)
                m_en = rank_eb >= 1
                plsc.store_scatter(en_m, [row, rank_eb - 1], ebv, mask=m_en)
                plsc.store_scatter(en_e, [row, rank_eb - 1], ebe, mask=m_en)
            for c in range(4):
                plsc.store_scatter(st_m, [row, rank_s[c]], sva[c], mask=jmask[c])
                plsc.store_scatter(st_e, [row, rank_s[c]], sve[c], mask=jmask[c])
                plsc.store_scatter(en_m, [row, rank_s[c] - 1], sva[c], mask=jmask[c])
                plsc.store_scatter(en_e, [row, rank_s[c] - 1], sve[c], mask=jmask[c])
            eb64 = eb_v[k, S]
            en_m[k, NOUT - 1] = eb64
            en_e[k, NOUT - 1] = near + eb64 * scale
            return rcarry

        lax.fori_loop(0, K, ray_body, 0, unroll=False)

        pltpu.sync_copy(st_e, o_se.at[pl.ds(base, K), :])
        pltpu.sync_copy(en_e, o_ee.at[pl.ds(base, K), :])
        pltpu.sync_copy(st_m, o_sm.at[pl.ds(base, K), :])
        pltpu.sync_copy(en_m, o_em.at[pl.ds(base, K), :])
        return carry

    lax.fori_loop(0, NCHUNK, chunk_body, 0, unroll=False)


@jax.jit
def _run(w2, eb, n1, f1, u):
    f32 = jnp.float32
    mesh = plsc.VectorSubcoreMesh(core_axis_name="c", subcore_axis_name="s")
    out_type = [jax.ShapeDtypeStruct((R, NOUT), f32) for _ in range(4)]
    scratch = [
        pltpu.VMEM((K, S), f32),
        pltpu.VMEM((K, S + 1), f32),
        pltpu.VMEM((K,), f32),
        pltpu.VMEM((K,), f32),
        pltpu.VMEM((S,), f32),
        pltpu.VMEM((S,), f32),
        pltpu.VMEM((S,), jnp.int32),
        pltpu.VMEM((S,), jnp.int32),
        pltpu.VMEM((K, NOUT), f32),
        pltpu.VMEM((K, NOUT), f32),
        pltpu.VMEM((K, NOUT), f32),
        pltpu.VMEM((K, NOUT), f32),
    ]
    kfn = functools.partial(
        pl.kernel, mesh=mesh, out_type=out_type, scratch_types=scratch,
    )(_body)
    return kfn(w2, eb, n1, f1, u)


def kernel(weights, existing_bins, nears, fars):
    w2 = weights[..., 0]
    n1 = nears[:, 0]
    f1 = fars[:, 0]
    u = jnp.linspace(0.0, 1.0 - 1.0 / NS, NS, dtype=jnp.float32) + jnp.float32(
        1.0 / (2 * NS))
    u = jnp.concatenate([u, jnp.zeros((S - NS,), jnp.float32)])
    se, ee, sm, em = _run(w2, existing_bins, n1, f1, u)
    return (se[..., None], ee[..., None], sm[..., None], em[..., None])


# speedup vs baseline: 7.4093x; 7.4093x over previous
import functools

import jax
import jax.numpy as jnp
from jax import lax
from jax.experimental import pallas as pl
from jax.experimental.pallas import tpu as pltpu
from jax.experimental.pallas import tpu_sc as plsc

R = 131072
S = 64
NS = 49
NOUT = 113
HIST_PAD = 0.01
EPS = 1e-05

_info = plsc.get_sparse_core_info()
NCORES = _info.num_cores
NSUB = _info.num_subcores
NWORK = NCORES * NSUB
RAYS_PER_W = R // NWORK
K = 64
NCHUNK = RAYS_PER_W // K


def _body(w_hbm, eb_hbm, nr_hbm, fr_hbm, u_hbm,
          o_se, o_ee, o_sm, o_em,
          w_v, eb_v, nr_v, fr_v, u_v, cdf_v, a_v, h2_v,
          st_e, en_e, st_m, en_m):
    wid = lax.axis_index("s") * NCORES + lax.axis_index("c")
    lanes = lax.iota(jnp.int32, 16)
    ones_i = jnp.ones((16,), jnp.int32)
    zeros_i = jnp.zeros((16,), jnp.int32)

    pltpu.sync_copy(u_hbm, u_v)
    u_vec = [u_v[pl.ds(16 * c, 16)] for c in range(4)]
    jvec = [lanes + 16 * c for c in range(4)]
    ivec = jvec
    jmask = [None, None, None, jvec[3] < NS]

    def chunk_body(cix, carry):
        base = wid * RAYS_PER_W + cix * K
        pltpu.sync_copy(w_hbm.at[pl.ds(base, K), :], w_v)
        pltpu.sync_copy(eb_hbm.at[pl.ds(base, K), :], eb_v)
        pltpu.sync_copy(nr_hbm.at[pl.ds(base, K)], nr_v)
        pltpu.sync_copy(fr_hbm.at[pl.ds(base, K)], fr_v)

        def ray_body(k, rcarry):
            row = lax.broadcast(k, (16,))
            w = [w_v[k, pl.ds(16 * c, 16)] + HIST_PAD for c in range(4)]
            s_pre = [jnp.sum(w[c]) for c in range(4)]
            total = (s_pre[0] + s_pre[1]) + (s_pre[2] + s_pre[3])
            padding = jnp.maximum(EPS - total, 0.0)
            wadj = padding * (1.0 / S)
            wsum = total + padding
            inv = jnp.ones((16,), jnp.float32) / lax.broadcast(wsum, (16,))
            carry_s = 0.0
            cdf = []
            for c in range(4):
                wc = w[c] + wadj
                cum = plsc.cumsum(wc) + carry_s
                carry_s = carry_s + s_pre[c] + 16.0 * wadj
                cdf.append(jnp.minimum(1.0, cum * inv))
                cdf_v[pl.ds(16 * c, 16)] = cdf[c]
            a_v[pl.ds(0, 16)] = jnp.where(lanes == 0, 1, 0)
            for c in range(1, 4):
                a_v[pl.ds(16 * c, 16)] = zeros_i
            for c in range(4):
                x = jnp.float32(NS) * cdf[c] - 0.5
                ti = x.astype(jnp.int32)
                j0 = ti + (ti.astype(jnp.float32) < x).astype(jnp.int32)
                j0 = jnp.clip(j0, 0, NS)
                plsc.addupdate_scatter(a_v, [j0], ones_i)
            carry_i = jnp.int32(0)
            below = []
            for c in range(4):
                av = a_v[pl.ds(16 * c, 16)]
                bc = plsc.cumsum(av) + carry_i - 1
                carry_i = carry_i + jnp.sum(av)
                below.append(jnp.minimum(bc, S - 1))
            near = plsc.load_gather(nr_v, [row])
            far = plsc.load_gather(fr_v, [row])
            scale = far - near
            h2_v[pl.ds(0, 16)] = zeros_i
            for c in range(1, 4):
                h2_v[pl.ds(16 * c, 16)] = zeros_i
            sva = []
            sve = []
            rank_s = []
            for c in range(4):
                b = below[c]
                g0 = plsc.load_gather(cdf_v, [jnp.maximum(b - 1, 0)])
                cdf_g0 = jnp.where(b == 0, 0.0, g0)
                cdf_g1 = plsc.load_gather(cdf_v, [b])
                bins_g0 = plsc.load_gather(eb_v, [row, b])
                bins_g1 = plsc.load_gather(eb_v, [row, b + 1])
                t = (u_vec[c] - cdf_g0) / (cdf_g1 - cdf_g0)
                t = jnp.clip(t, 0.0, 1.0)
                sv = bins_g0 + t * (bins_g1 - bins_g0)
                sva.append(sv)
                sve.append(near + sv * scale)
                rank_s.append(jvec[c] + b + 1)
                plsc.addupdate_scatter(h2_v, [b], ones_i, mask=jmask[c])
            carry_i = jnp.int32(0)
            for c in range(4):
                hv = h2_v[pl.ds(16 * c, 16)]
                excl = plsc.cumsum(hv) - hv + carry_i
                carry_i = carry_i + jnp.sum(hv)
                rank_eb = ivec[c] + excl
                ebv = eb_v[k, pl.ds(16 * c, 16)]
                ebe = near + ebv * scale
                plsc.store_scatter(st_m, [row, rank_eb], ebv)
                plsc.store_scatter(st_e, [row, rank_eb], ebe)
                m_en = rank_eb >= 1
                plsc.store_scatter(en_m, [row, rank_eb - 1], ebv, mask=m_en)
                plsc.store_scatter(en_e, [row, rank_eb - 1], ebe, mask=m_en)
            for c in range(4):
                plsc.store_scatter(st_m, [row, rank_s[c]], sva[c], mask=jmask[c])
                plsc.store_scatter(st_e, [row, rank_s[c]], sve[c], mask=jmask[c])
                plsc.store_scatter(en_m, [row, rank_s[c] - 1], sva[c], mask=jmask[c])
                plsc.store_scatter(en_e, [row, rank_s[c] - 1], sve[c], mask=jmask[c])
            eb64 = plsc.load_gather(eb_v, [row, jnp.full((16,), S, jnp.int32)])
            last_col = jnp.full((16,), NOUT - 1, jnp.int32)
            lane0 = lanes < 1
            plsc.store_scatter(en_m, [row, last_col], eb64, mask=lane0)
            plsc.store_scatter(en_e, [row, last_col], near + eb64 * scale,
                               mask=lane0)
            return rcarry

        lax.fori_loop(0, K, ray_body, 0, unroll=False)

        pltpu.sync_copy(st_e, o_se.at[pl.ds(base, K), :])
        pltpu.sync_copy(en_e, o_ee.at[pl.ds(base, K), :])
        pltpu.sync_copy(st_m, o_sm.at[pl.ds(base, K), :])
        pltpu.sync_copy(en_m, o_em.at[pl.ds(base, K), :])
        return carry

    lax.fori_loop(0, NCHUNK, chunk_body, 0, unroll=False)


@jax.jit
def _run(w2, eb, n1, f1, u):
    f32 = jnp.float32
    mesh = plsc.VectorSubcoreMesh(core_axis_name="c", subcore_axis_name="s")
    out_type = [jax.ShapeDtypeStruct((R, NOUT), f32) for _ in range(4)]
    scratch = [
        pltpu.VMEM((K, S), f32),
        pltpu.VMEM((K, S + 1), f32),
        pltpu.VMEM((K,), f32),
        pltpu.VMEM((K,), f32),
        pltpu.VMEM((S,), f32),
        pltpu.VMEM((S,), f32),
        pltpu.VMEM((S,), jnp.int32),
        pltpu.VMEM((S,), jnp.int32),
        pltpu.VMEM((K, NOUT), f32),
        pltpu.VMEM((K, NOUT), f32),
        pltpu.VMEM((K, NOUT), f32),
        pltpu.VMEM((K, NOUT), f32),
    ]
    kfn = functools.partial(
        pl.kernel, mesh=mesh, out_type=out_type, scratch_types=scratch,
        compiler_params=pltpu.CompilerParams(needs_layout_passes=False),
    )(_body)
    return kfn(w2, eb, n1, f1, u)


def kernel(weights, existing_bins, nears, fars):
    w2 = weights[..., 0]
    n1 = nears[:, 0]
    f1 = fars[:, 0]
    u = jnp.linspace(0.0, 1.0 - 1.0 / NS, NS, dtype=jnp.float32) + jnp.float32(
        1.0 / (2 * NS))
    u = jnp.concatenate([u, jnp.zeros((S - NS,), jnp.float32)])
    se, ee, sm, em = _run(w2, existing_bins, n1, f1, u)
    return (se[..., None], ee[..., None], sm[..., None], em[..., None])
